# baseline (speedup 1.0000x reference)
import functools

import jax
import jax.numpy as jnp
from jax import lax
from jax.experimental import pallas as pl
from jax.experimental.pallas import tpu as pltpu
from jax.experimental.pallas import tpu_sc as plsc

C = 8
N = 32768
TIME = 512
BATCH = 8192

NC = 2
NS = 16
NW = NC * NS
ROWS = C * BATCH
RPW = ROWS // NW
WPC = NW // C
RAW = RPW + TIME

_mesh = plsc.VectorSubcoreMesh(core_axis_name="c", subcore_axis_name="s")


@functools.partial(
    pl.kernel,
    out_type=jax.ShapeDtypeStruct((C, BATCH, TIME), jnp.float32),
    mesh=_mesh,
    scratch_types=[
        pltpu.VMEM((RAW,), jnp.float32),
        pltpu.VMEM((8, TIME), jnp.float32),
        pltpu.VMEM((8, TIME), jnp.float32),
        pltpu.SemaphoreType.DMA,
        pltpu.SemaphoreType.DMA,
        pltpu.SemaphoreType.DMA,
    ],
    compiler_params=pltpu.CompilerParams(use_tc_tiling_on_sc=True),
)
def _hankel_sc(x_hbm, out_hbm, raw_v, buf0_v, buf1_v, in_sem, sem0, sem1):
    wid = lax.axis_index("s") * NC + lax.axis_index("c")
    chan = wid // WPC
    b0 = (wid % WPC) * RPW
    base = chan * N + b0

    pltpu.async_copy(x_hbm.at[pl.ds(base, RAW)], raw_v, in_sem).wait()

    def build(buf, off):
        for i in range(8):
            for m in range(TIME // 16):
                buf[i, pl.ds(16 * m, 16)] = raw_v[pl.ds(off + i + 16 * m, 16)]

    def body(q2, carry):
        off = 16 * q2
        row = b0 + off

        @pl.when(q2 > 0)
        def _():
            pltpu.make_async_copy(
                buf0_v, out_hbm.at[chan, pl.ds(b0, 8)], sem0
            ).wait()

        build(buf0_v, off)
        pltpu.async_copy(buf0_v, out_hbm.at[chan, pl.ds(row, 8)], sem0)

        @pl.when(q2 > 0)
        def _():
            pltpu.make_async_copy(
                buf1_v, out_hbm.at[chan, pl.ds(b0, 8)], sem1
            ).wait()

        build(buf1_v, off + 8)
        pltpu.async_copy(buf1_v, out_hbm.at[chan, pl.ds(row + 8, 8)], sem1)
        return carry

    lax.fori_loop(0, RPW // 16, body, 0)

    pltpu.make_async_copy(buf0_v, out_hbm.at[chan, pl.ds(b0, 8)], sem0).wait()
    pltpu.make_async_copy(buf1_v, out_hbm.at[chan, pl.ds(b0, 8)], sem1).wait()


def kernel(x):
    return _hankel_sc(x.reshape(-1))


# speedup vs baseline: 1.8881x; 1.8881x over previous
import functools

import jax
import jax.numpy as jnp
from jax import lax
from jax.experimental import pallas as pl
from jax.experimental.pallas import tpu as pltpu
from jax.experimental.pallas import tpu_sc as plsc

C = 8
N = 32768
TIME = 512
BATCH = 8192

NC = 2
NS = 16
NW = NC * NS
ROWS = C * BATCH
RPW = ROWS // NW
WPC = NW // C

SHC = 128
CHUNK = 512
NCHUNK = RPW // CHUNK
CWLEN = (CHUNK - SHC) + TIME
NQ = CHUNK // SHC
NL = CWLEN // SHC
RAWC = CHUNK + TIME

_mesh = plsc.VectorSubcoreMesh(core_axis_name="c", subcore_axis_name="s")


@functools.partial(
    pl.kernel,
    out_type=jax.ShapeDtypeStruct((C, BATCH, TIME), jnp.float32),
    mesh=_mesh,
    scratch_types=[
        pltpu.VMEM((RAWC,), jnp.float32),
        pltpu.VMEM((SHC, CWLEN), jnp.float32),
        pltpu.SemaphoreType.DMA,
        pltpu.SemaphoreType.DMA,
        pltpu.SemaphoreType.DMA,
    ],
    compiler_params=pltpu.CompilerParams(use_tc_tiling_on_sc=True),
)
def _hankel_sc(x_hbm, out_hbm, raw_v, win_v, in_sem, sem_a, sem_b):
    wid = lax.axis_index("s") * NC + lax.axis_index("c")
    chan = wid // WPC
    b0 = (wid % WPC) * RPW
    base = chan * N + b0

    def drain(sem, count):
        def wait(i, carry):
            pltpu.make_async_copy(
                win_v.at[pl.ds(0, SHC), pl.ds(0, TIME)],
                out_hbm.at[chan, pl.ds(b0, SHC)],
                sem,
            ).wait()
            return carry

        lax.fori_loop(0, count, wait, 0)

    for ck in range(NCHUNK):
        eb = base + CHUNK * ck
        pltpu.async_copy(x_hbm.at[pl.ds(eb, RAWC)], raw_v, in_sem).wait()

        for L in range(NL):
            if ck > 0 and L == 0:
                drain(sem_a, NQ - 1)
            if ck > 0 and L == NQ - 1:
                drain(sem_b, 1)

            def build(j, carry):
                for m in range(SHC // 16):
                    win_v[j, pl.ds(128 * L + 16 * m, 16)] = raw_v[
                        pl.ds(j + 128 * L + 16 * m, 16)
                    ]
                return carry

            lax.fori_loop(0, SHC, build, 0)

            if L >= NL - NQ:
                qq = L - (NL - NQ)
                pltpu.async_copy(
                    win_v.at[pl.ds(0, SHC), pl.ds(128 * qq, TIME)],
                    out_hbm.at[chan, pl.ds(b0 + CHUNK * ck + SHC * qq, SHC)],
                    sem_b if qq == NQ - 1 else sem_a,
                )

    drain(sem_a, NQ - 1)
    drain(sem_b, 1)


def kernel(x):
    return _hankel_sc(x.reshape(-1))


# speedup vs baseline: 1.9398x; 1.0273x over previous
import functools

import jax
import jax.numpy as jnp
from jax import lax
from jax.experimental import pallas as pl
from jax.experimental.pallas import tpu as pltpu
from jax.experimental.pallas import tpu_sc as plsc

C = 8
N = 32768
TIME = 512
BATCH = 8192

NC = 2
NS = 16
NW = NC * NS
ROWS = C * BATCH
RPW = ROWS // NW
WPC = NW // C

SHC = 128
CHUNK = 512
NCHUNK = RPW // CHUNK
CWLEN = (CHUNK - SHC) + TIME
NQ = CHUNK // SHC
NL = CWLEN // SHC
RAWC = CHUNK + TIME

_mesh = plsc.VectorSubcoreMesh(core_axis_name="c", subcore_axis_name="s")


@functools.partial(
    pl.kernel,
    out_type=jax.ShapeDtypeStruct((C, BATCH, TIME), jnp.float32),
    mesh=_mesh,
    scratch_types=[
        pltpu.VMEM((RAWC,), jnp.float32),
        pltpu.VMEM((RAWC,), jnp.float32),
        pltpu.VMEM((SHC, CWLEN), jnp.float32),
        pltpu.SemaphoreType.DMA,
        pltpu.SemaphoreType.DMA,
        pltpu.SemaphoreType.DMA,
        pltpu.SemaphoreType.DMA,
        pltpu.SemaphoreType.DMA,
    ],
    compiler_params=pltpu.CompilerParams(use_tc_tiling_on_sc=True),
)
def _hankel_sc(x_hbm, out_hbm, raw_a, raw_b, win_v, in_sem, s0, s1, s2, s3):
    qsem = (s0, s1, s2, s3)
    raws = (raw_a, raw_b)
    wid = lax.axis_index("s") * NC + lax.axis_index("c")
    chan = wid // WPC
    b0 = (wid % WPC) * RPW
    base = chan * N + b0

    def drain(sem):
        pltpu.make_async_copy(
            win_v.at[pl.ds(0, SHC), pl.ds(0, TIME)],
            out_hbm.at[chan, pl.ds(b0, SHC)],
            sem,
        ).wait()

    pltpu.async_copy(x_hbm.at[pl.ds(base, RAWC)], raw_a, in_sem)

    for ck in range(NCHUNK):
        rawc = raws[ck % 2]
        pltpu.make_async_copy(
            x_hbm.at[pl.ds(base, RAWC)], rawc, in_sem
        ).wait()
        if ck + 1 < NCHUNK:
            pltpu.async_copy(
                x_hbm.at[pl.ds(base + CHUNK * (ck + 1), RAWC)],
                raws[(ck + 1) % 2],
                in_sem,
            )

        for L in range(NL):
            if ck > 0 and L < NQ:
                drain(qsem[L])

            def build(j, carry):
                for m in range(SHC // 16):
                    win_v[j, pl.ds(128 * L + 16 * m, 16)] = rawc[
                        pl.ds(j + 128 * L + 16 * m, 16)
                    ]
                return carry

            lax.fori_loop(0, SHC, build, 0)

            if L >= NL - NQ:
                qq = L - (NL - NQ)
                pltpu.async_copy(
                    win_v.at[pl.ds(0, SHC), pl.ds(128 * qq, TIME)],
                    out_hbm.at[chan, pl.ds(b0 + CHUNK * ck + SHC * qq, SHC)],
                    qsem[qq],
                )

    for qq in range(NQ):
        drain(qsem[qq])


def kernel(x):
    return _hankel_sc(x.reshape(-1))


# speedup vs baseline: 3.3411x; 1.7224x over previous
import functools

import jax
import jax.numpy as jnp
from jax import lax
from jax.experimental import pallas as pl
from jax.experimental.pallas import tpu as pltpu
from jax.experimental.pallas import tpu_sc as plsc

C = 8
N = 32768
TIME = 512
BATCH = 8192

NC = 2
NS = 16
NW = NC * NS
ROWS = C * BATCH
RPW = ROWS // NW
WPC = NW // C

SHC = 128
CHUNK = 512
NCHUNK = RPW // CHUNK
CWLEN = (CHUNK - SHC) + TIME
NQ = CHUNK // SHC
NL = CWLEN // SHC
RAWC = CHUNK + TIME

_mesh = plsc.VectorSubcoreMesh(core_axis_name="c", subcore_axis_name="s")


@functools.partial(
    pl.kernel,
    out_type=jax.ShapeDtypeStruct((C, BATCH, TIME), jnp.float32),
    mesh=_mesh,
    scratch_types=[
        pltpu.VMEM((RAWC,), jnp.float32),
        pltpu.VMEM((RAWC,), jnp.float32),
        pltpu.VMEM((SHC, CWLEN), jnp.float32),
        pltpu.SemaphoreType.DMA,
        pltpu.SemaphoreType.DMA,
        pltpu.SemaphoreType.DMA,
        pltpu.SemaphoreType.DMA,
        pltpu.SemaphoreType.DMA,
    ],
    compiler_params=pltpu.CompilerParams(use_tc_tiling_on_sc=True),
)
def _hankel_sc(x_hbm, out_hbm, raw_a, raw_b, win_v, in_sem, s0, s1, s2, s3):
    qsem = (s0, s1, s2, s3)
    raws = (raw_a, raw_b)
    wid = lax.axis_index("s") * NC + lax.axis_index("c")
    chan = wid // WPC
    b0 = (wid % WPC) * RPW
    base = chan * N + b0

    def drain(sem):
        pltpu.make_async_copy(
            win_v.at[pl.ds(0, SHC), pl.ds(0, TIME)],
            out_hbm.at[chan, pl.ds(b0, SHC)],
            sem,
        ).wait()

    pltpu.async_copy(x_hbm.at[pl.ds(base, RAWC)], raw_a, in_sem)

    for ck in range(NCHUNK):
        rawc = raws[ck % 2]
        pltpu.make_async_copy(
            x_hbm.at[pl.ds(base, RAWC)], rawc, in_sem
        ).wait()
        if ck + 1 < NCHUNK:
            pltpu.async_copy(
                x_hbm.at[pl.ds(base + CHUNK * (ck + 1), RAWC)],
                raws[(ck + 1) % 2],
                in_sem,
            )

        for L in range(NL):
            if ck > 0 and L < NQ:
                drain(qsem[L])

            @plsc.parallel_loop(0, SHC, unroll=4)
            def build(j):
                for m in range(SHC // 16):
                    win_v[j, pl.ds(128 * L + 16 * m, 16)] = rawc[
                        pl.ds(j + 128 * L + 16 * m, 16)
                    ]

            if L >= NL - NQ:
                qq = L - (NL - NQ)
                pltpu.async_copy(
                    win_v.at[pl.ds(0, SHC), pl.ds(128 * qq, TIME)],
                    out_hbm.at[chan, pl.ds(b0 + CHUNK * ck + SHC * qq, SHC)],
                    qsem[qq],
                )

    for qq in range(NQ):
        drain(qsem[qq])


def kernel(x):
    return _hankel_sc(x.reshape(-1))
